# baseline (speedup 1.0000x reference)
import functools

import jax
import jax.numpy as jnp
from jax import lax
from jax.experimental import pallas as pl
from jax.experimental.pallas import tpu as pltpu
from jax.experimental.pallas import tpu_sc as plsc

HIDDEN = 2048
NUM_EXPERTS = 16
TOP_K = 2
EPS = 1e-6
TOKENS = 8192

BLOCK_T = 512

_SC_INFO = plsc.get_sparse_core_info()
_NC, _NS, _NL = _SC_INFO.num_cores, _SC_INFO.num_subcores, _SC_INFO.num_lanes
_NW = _NC * _NS
_TOK_W = TOKENS // _NW
_GROUPS = _TOK_W // _NL


def _dense_block(x_ref, scale_ref, w_ref, p_ref):
    xb = x_ref[...]
    ms = jnp.mean(xb * xb, axis=-1, keepdims=True)
    y = xb * lax.rsqrt(ms + EPS)
    y = y * scale_ref[...]
    y = y * (HIDDEN ** -0.5)
    logits = lax.dot_general(
        y, w_ref[...],
        dimension_numbers=(((1,), (1,)), ((), ())),
        preferred_element_type=jnp.float32,
    )
    m = jnp.max(logits, axis=-1, keepdims=True)
    e = jnp.exp(logits - m)
    p_ref[...] = e / jnp.sum(e, axis=-1, keepdims=True)


def _sc_router(probs_flat_hbm, pes_hbm, i1_hbm, i2_hbm, w1_hbm, w2_hbm,
               pv, pesv, i1v, i2v, w1v, w2v):
    wid = lax.axis_index("s") * _NC + lax.axis_index("c")
    base = wid * _TOK_W
    pltpu.sync_copy(probs_flat_hbm.at[pl.ds(base * NUM_EXPERTS,
                                            _TOK_W * NUM_EXPERTS)], pv)
    pltpu.sync_copy(pes_hbm, pesv)
    lanes = lax.iota(jnp.int32, _NL)
    for g in range(_GROUPS):
        goff = g * _NL * NUM_EXPERTS
        m1 = jnp.full((_NL,), -jnp.inf, jnp.float32)
        m2 = jnp.full((_NL,), -jnp.inf, jnp.float32)
        i1 = jnp.zeros((_NL,), jnp.int32)
        i2 = jnp.zeros((_NL,), jnp.int32)
        for e in range(NUM_EXPERTS):
            v = plsc.load_gather(pv, [lanes * NUM_EXPERTS + (goff + e)])
            is1 = v > m1
            is2 = jnp.logical_not(is1) & (v > m2)
            m2 = jnp.where(is1, m1, jnp.where(is2, v, m2))
            i2 = jnp.where(is1, i1, jnp.where(is2, e, i2))
            m1 = jnp.where(is1, v, m1)
            i1 = jnp.where(is1, e, i1)
        g1 = plsc.load_gather(pesv, [i1])
        g2 = plsc.load_gather(pesv, [i2])
        s = m1 + m2
        sl = pl.ds(g * _NL, _NL)
        i1v[sl] = i1
        i2v[sl] = i2
        w1v[sl] = (m1 / s) * g1
        w2v[sl] = (m2 / s) * g2
    out_sl = pl.ds(base, _TOK_W)
    pltpu.sync_copy(i1v, i1_hbm.at[out_sl])
    pltpu.sync_copy(i2v, i2_hbm.at[out_sl])
    pltpu.sync_copy(w1v, w1_hbm.at[out_sl])
    pltpu.sync_copy(w2v, w2_hbm.at[out_sl])


_sc_router_call = functools.partial(
    pl.kernel,
    out_type=[
        jax.ShapeDtypeStruct((TOKENS,), jnp.int32),
        jax.ShapeDtypeStruct((TOKENS,), jnp.int32),
        jax.ShapeDtypeStruct((TOKENS,), jnp.float32),
        jax.ShapeDtypeStruct((TOKENS,), jnp.float32),
    ],
    mesh=plsc.VectorSubcoreMesh(core_axis_name="c", subcore_axis_name="s"),
    compiler_params=pltpu.CompilerParams(needs_layout_passes=False),
    scratch_types=[
        pltpu.VMEM((_TOK_W * NUM_EXPERTS,), jnp.float32),
        pltpu.VMEM((NUM_EXPERTS,), jnp.float32),
        pltpu.VMEM((_TOK_W,), jnp.int32),
        pltpu.VMEM((_TOK_W,), jnp.int32),
        pltpu.VMEM((_TOK_W,), jnp.float32),
        pltpu.VMEM((_TOK_W,), jnp.float32),
    ],
)(_sc_router)


@jax.jit
def kernel(x, scale, per_expert_scale, W_proj):
    probs = pl.pallas_call(
        _dense_block,
        grid=(TOKENS // BLOCK_T,),
        in_specs=[
            pl.BlockSpec((BLOCK_T, HIDDEN), lambda i: (i, 0)),
            pl.BlockSpec((1, HIDDEN), lambda i: (0, 0)),
            pl.BlockSpec((NUM_EXPERTS, HIDDEN), lambda i: (0, 0)),
        ],
        out_specs=pl.BlockSpec((BLOCK_T, NUM_EXPERTS), lambda i: (i, 0)),
        out_shape=jax.ShapeDtypeStruct((TOKENS, NUM_EXPERTS), jnp.float32),
    )(x, scale.reshape(1, HIDDEN), W_proj)
    i1, i2, w1, w2 = _sc_router_call(probs.reshape(-1), per_expert_scale)
    idx = jnp.stack([i1, i2], axis=1).astype(jnp.int64)
    wgt = jnp.stack([w1, w2], axis=1)
    return idx, wgt


# speedup vs baseline: 1.0692x; 1.0692x over previous
import functools

import jax
import jax.numpy as jnp
from jax import lax
from jax.experimental import pallas as pl
from jax.experimental.pallas import tpu as pltpu
from jax.experimental.pallas import tpu_sc as plsc

HIDDEN = 2048
NUM_EXPERTS = 16
TOP_K = 2
EPS = 1e-6
TOKENS = 8192

BLOCK_T = 1024

_SC_INFO = plsc.get_sparse_core_info()
_NC, _NS, _NL = _SC_INFO.num_cores, _SC_INFO.num_subcores, _SC_INFO.num_lanes
_NW = _NC * _NS
_TOK_W = TOKENS // _NW
_GROUPS = _TOK_W // _NL


def _dense_block(x_ref, scale_ref, w_ref, p_ref):
    xb = x_ref[...]
    ms = jnp.mean(xb * xb, axis=-1, keepdims=True)
    y = xb * lax.rsqrt(ms + EPS)
    y = y * scale_ref[...]
    y = y * (HIDDEN ** -0.5)
    logits = lax.dot_general(
        y, w_ref[...],
        dimension_numbers=(((1,), (1,)), ((), ())),
        preferred_element_type=jnp.float32,
    )
    m = jnp.max(logits, axis=-1, keepdims=True)
    e = jnp.exp(logits - m)
    p_ref[...] = e / jnp.sum(e, axis=-1, keepdims=True)


def _sc_router(probs_flat_hbm, pes_hbm, i1_hbm, i2_hbm, w1_hbm, w2_hbm,
               pv, pesv, i1v, i2v, w1v, w2v):
    wid = lax.axis_index("s") * _NC + lax.axis_index("c")
    base = wid * _TOK_W
    pltpu.sync_copy(probs_flat_hbm.at[pl.ds(base * NUM_EXPERTS,
                                            _TOK_W * NUM_EXPERTS)], pv)
    pltpu.sync_copy(pes_hbm, pesv)
    lanes = lax.iota(jnp.int32, _NL)
    for g in range(_GROUPS):
        goff = g * _NL * NUM_EXPERTS
        m1 = jnp.full((_NL,), -jnp.inf, jnp.float32)
        m2 = jnp.full((_NL,), -jnp.inf, jnp.float32)
        i1 = jnp.zeros((_NL,), jnp.int32)
        i2 = jnp.zeros((_NL,), jnp.int32)
        for e in range(NUM_EXPERTS):
            v = plsc.load_gather(pv, [lanes * NUM_EXPERTS + (goff + e)])
            is1 = v > m1
            is2 = jnp.logical_not(is1) & (v > m2)
            m2 = jnp.where(is1, m1, jnp.where(is2, v, m2))
            i2 = jnp.where(is1, i1, jnp.where(is2, e, i2))
            m1 = jnp.where(is1, v, m1)
            i1 = jnp.where(is1, e, i1)
        g1 = plsc.load_gather(pesv, [i1])
        g2 = plsc.load_gather(pesv, [i2])
        s = m1 + m2
        sl = pl.ds(g * _NL, _NL)
        i1v[sl] = i1
        i2v[sl] = i2
        w1v[sl] = (m1 / s) * g1
        w2v[sl] = (m2 / s) * g2
    out_sl = pl.ds(base, _TOK_W)
    pltpu.sync_copy(i1v, i1_hbm.at[out_sl])
    pltpu.sync_copy(i2v, i2_hbm.at[out_sl])
    pltpu.sync_copy(w1v, w1_hbm.at[out_sl])
    pltpu.sync_copy(w2v, w2_hbm.at[out_sl])


_sc_router_call = functools.partial(
    pl.kernel,
    out_type=[
        jax.ShapeDtypeStruct((TOKENS,), jnp.int32),
        jax.ShapeDtypeStruct((TOKENS,), jnp.int32),
        jax.ShapeDtypeStruct((TOKENS,), jnp.float32),
        jax.ShapeDtypeStruct((TOKENS,), jnp.float32),
    ],
    mesh=plsc.VectorSubcoreMesh(core_axis_name="c", subcore_axis_name="s"),
    compiler_params=pltpu.CompilerParams(needs_layout_passes=False),
    scratch_types=[
        pltpu.VMEM((_TOK_W * NUM_EXPERTS,), jnp.float32),
        pltpu.VMEM((NUM_EXPERTS,), jnp.float32),
        pltpu.VMEM((_TOK_W,), jnp.int32),
        pltpu.VMEM((_TOK_W,), jnp.int32),
        pltpu.VMEM((_TOK_W,), jnp.float32),
        pltpu.VMEM((_TOK_W,), jnp.float32),
    ],
)(_sc_router)


@jax.jit
def kernel(x, scale, per_expert_scale, W_proj):
    probs = pl.pallas_call(
        _dense_block,
        grid=(TOKENS // BLOCK_T,),
        in_specs=[
            pl.BlockSpec((BLOCK_T, HIDDEN), lambda i: (i, 0)),
            pl.BlockSpec((1, HIDDEN), lambda i: (0, 0)),
            pl.BlockSpec((NUM_EXPERTS, HIDDEN), lambda i: (0, 0)),
        ],
        out_specs=pl.BlockSpec((BLOCK_T, NUM_EXPERTS), lambda i: (i, 0)),
        out_shape=jax.ShapeDtypeStruct((TOKENS, NUM_EXPERTS), jnp.float32),
    )(x, scale.reshape(1, HIDDEN), W_proj)
    i1, i2, w1, w2 = _sc_router_call(probs.reshape(-1), per_expert_scale)
    idx = jnp.stack([i1, i2], axis=1).astype(jnp.int64)
    wgt = jnp.stack([w1, w2], axis=1)
    return idx, wgt
